# baseline (speedup 1.0000x reference)
import jax
import jax.numpy as jnp
from jax import lax
from jax.experimental import pallas as pl
from jax.experimental.pallas import tpu as pltpu
from jax.experimental.pallas import tpu_sc as plsc

NUM_CORES = 2
NUM_SUBCORES = 16
NUM_WORKERS = NUM_CORES * NUM_SUBCORES
LANES = 16

ROWS = 4 * 8192
D = 1024
ROWS_PER_WORKER = ROWS // NUM_WORKERS
CHUNK = 16
NCHUNKS = ROWS_PER_WORKER // CHUNK
DEPTH = 4


def _body(x_hbm, keep_hbm, out_hbm, keep_v, zrow_v, csems, zsem):
    wid = lax.axis_index("s") * NUM_CORES + lax.axis_index("c")
    base = wid * ROWS_PER_WORKER

    pltpu.sync_copy(keep_hbm.at[pl.ds(base, ROWS_PER_WORKER)], keep_v)

    zeros = jnp.zeros((LANES,), jnp.float32)
    for j in range(D // LANES):
        zrow_v[0, pl.ds(j * LANES, LANES)] = zeros

    def copy_desc(g, slot):
        return pltpu.make_async_copy(
            x_hbm.at[pl.ds(base + g * CHUNK, CHUNK)],
            out_hbm.at[pl.ds(base + g * CHUNK, CHUNK)],
            csems.at[slot])

    def zero_desc(row):
        return pltpu.make_async_copy(
            zrow_v, out_hbm.at[pl.ds(base + row, 1)], zsem)

    def zeros_pass(g, start):
        kvec = keep_v[pl.ds(g * CHUNK, CHUNK)]
        for r in range(CHUNK):
            @pl.when(kvec[r] == 0.0)
            def _(r=r):
                if start:
                    zero_desc(g * CHUNK + r).start()
                else:
                    zero_desc(g * CHUNK + r).wait()

    for i in range(DEPTH):
        copy_desc(i, i).start()

    def group_body(go, _):
        for i in range(DEPTH):
            g = go * DEPTH + i

            @pl.when(g + DEPTH < NCHUNKS)
            def _():
                copy_desc(g + DEPTH, i).start()

            copy_desc(g, i).wait()
            zeros_pass(g, start=True)

            @pl.when(g >= DEPTH)
            def _():
                zeros_pass(g - DEPTH, start=False)
        return 0

    lax.fori_loop(0, NCHUNKS // DEPTH, group_body, 0)

    for i in range(DEPTH):
        zeros_pass(NCHUNKS - DEPTH + i, start=False)


@jax.jit
def _masked_zero(x2d, keep):
    mesh = plsc.VectorSubcoreMesh(
        core_axis_name="c", subcore_axis_name="s",
        num_cores=NUM_CORES, num_subcores=NUM_SUBCORES)
    return pl.kernel(
        _body,
        out_type=jax.ShapeDtypeStruct((ROWS, D), jnp.float32),
        mesh=mesh,
        scratch_types=[
            pltpu.VMEM((ROWS_PER_WORKER,), jnp.float32),
            pltpu.VMEM((1, D), jnp.float32),
            pltpu.SemaphoreType.DMA((DEPTH,)),
            pltpu.SemaphoreType.DMA,
        ],
    )(x2d, keep)


def kernel(x, x_pad_mask):
    x2d = x.reshape(ROWS, D)
    keep = jnp.where(x_pad_mask.reshape(ROWS), 0.0, 1.0).astype(jnp.float32)
    out = _masked_zero(x2d, keep)
    return out.reshape(x.shape)


# speedup vs baseline: 37.1404x; 37.1404x over previous
import jax
import jax.numpy as jnp
from jax import lax
from jax.experimental import pallas as pl
from jax.experimental.pallas import tpu as pltpu
from jax.experimental.pallas import tpu_sc as plsc

NUM_CORES = 2
NUM_SUBCORES = 16
NUM_WORKERS = NUM_CORES * NUM_SUBCORES
LANES = 16

ROWS = 4 * 8192
D = 1024
ROWS_PER_WORKER = ROWS // NUM_WORKERS
CHUNK = 16
NCHUNKS = ROWS_PER_WORKER // CHUNK
NBUF = 4


def _body(x_hbm, keep_hbm, out_hbm, keep_v, zrow_v, spmem, in_sems, out_sems):
    sid = lax.axis_index("s")
    wid = sid * NUM_CORES + lax.axis_index("c")
    base = wid * ROWS_PER_WORKER

    pltpu.sync_copy(keep_hbm.at[pl.ds(base, ROWS_PER_WORKER)], keep_v)

    zeros = jnp.zeros((LANES,), jnp.float32)
    for j in range(D // LANES):
        zrow_v[0, pl.ds(j * LANES, LANES)] = zeros

    def start_in(g, slot):
        pltpu.make_async_copy(
            x_hbm.at[pl.ds(base + g * CHUNK, CHUNK)],
            spmem.at[sid, slot], in_sems.at[slot]).start()

    def wait_in(g, slot):
        pltpu.make_async_copy(
            x_hbm.at[pl.ds(base + g * CHUNK, CHUNK)],
            spmem.at[sid, slot], in_sems.at[slot]).wait()

    def start_out(g, slot):
        pltpu.make_async_copy(
            spmem.at[sid, slot],
            out_hbm.at[pl.ds(base + g * CHUNK, CHUNK)],
            out_sems.at[slot]).start()

    def wait_out(g, slot):
        pltpu.make_async_copy(
            spmem.at[sid, slot],
            out_hbm.at[pl.ds(base + g * CHUNK, CHUNK)],
            out_sems.at[slot]).wait()

    def zero_masked(g, slot):
        kvec = keep_v[pl.ds(g * CHUNK, CHUNK)]
        for r in range(CHUNK):
            @pl.when(kvec[r] == 0.0)
            def _(r=r):
                pltpu.sync_copy(zrow_v, spmem.at[sid, slot, pl.ds(r, 1)])

    start_in(0, 0)
    start_in(1, 1)

    def group_body(go, _):
        for i in range(NBUF):
            g = go * NBUF + i
            gp = g + 2
            slot_p = (i + 2) % NBUF

            @pl.when(gp < NCHUNKS)
            def _():
                @pl.when(gp >= NBUF)
                def _():
                    wait_out(gp - NBUF, slot_p)
                start_in(gp, slot_p)

            wait_in(g, i)
            zero_masked(g, i)
            start_out(g, i)
        return 0

    lax.fori_loop(0, NCHUNKS // NBUF, group_body, 0)

    wait_out(NCHUNKS - 2, (NCHUNKS - 2) % NBUF)
    wait_out(NCHUNKS - 1, (NCHUNKS - 1) % NBUF)


@jax.jit
def _masked_zero(x2d, keep):
    mesh = plsc.VectorSubcoreMesh(
        core_axis_name="c", subcore_axis_name="s",
        num_cores=NUM_CORES, num_subcores=NUM_SUBCORES)
    return pl.kernel(
        _body,
        out_type=jax.ShapeDtypeStruct((ROWS, D), jnp.float32),
        mesh=mesh,
        scratch_types=[
            pltpu.VMEM((ROWS_PER_WORKER,), jnp.float32),
            pltpu.VMEM((1, D), jnp.float32),
            pltpu.VMEM_SHARED((NUM_SUBCORES, NBUF, CHUNK, D), jnp.float32),
            pltpu.SemaphoreType.DMA((NBUF,)),
            pltpu.SemaphoreType.DMA((NBUF,)),
        ],
    )(x2d, keep)


def kernel(x, x_pad_mask):
    x2d = x.reshape(ROWS, D)
    keep = jnp.where(x_pad_mask.reshape(ROWS), 0.0, 1.0).astype(jnp.float32)
    out = _masked_zero(x2d, keep)
    return out.reshape(x.shape)
